# baseline (speedup 1.0000x reference)
import functools
import math

import jax
import jax.numpy as jnp
from jax import lax
from jax.experimental import pallas as pl
from jax.experimental.pallas import tpu as pltpu
from jax.experimental.pallas import tpu_sc as plsc

VOCAB = 1000000
EMB = 64
BATCH = 16384
NEG = 10

NC = 2
NS = 16
NW = NC * NS
ROWS_W = BATCH // NW
CHUNK = 64
NCHUNK = ROWS_W // CHUNK
NEG_CH = CHUNK * NEG
NEG_GATHERS = NEG_CH // 128


def _sc_scores_body(pos_hbm, neg_hbm, v_hbm, out_hbm,
                    sp_out, sn_out,
                    idx_c0, idx_c1, idx_p0, idx_p1, idx_n0, idx_n1,
                    rows_v0, rows_v1, rows_p0, rows_p1, rows_n0, rows_n1,
                    sp, sn, sem0, sem1):
    wid = lax.axis_index("s") * NC + lax.axis_index("c")
    idx_c = (idx_c0, idx_c1)
    idx_p = (idx_p0, idx_p1)
    idx_n = (idx_n0, idx_n1)
    rows_v = (rows_v0, rows_v1)
    rows_p = (rows_p0, rows_p1)
    rows_n = (rows_n0, rows_n1)
    sems = (sem0, sem1)
    iota16 = lax.broadcasted_iota(jnp.int32, (16,), 0)

    def issue(c, s):
        base = wid * ROWS_W + c * CHUNK
        for g in range(CHUNK // 16):
            idx_c[s][pl.ds(g * 16, 16)] = base + g * 16 + iota16
        pltpu.sync_copy(pos_hbm.at[pl.ds(base, CHUNK)], idx_p[s])
        pltpu.sync_copy(neg_hbm.at[pl.ds(base * NEG, NEG_CH)], idx_n[s])
        cps = [pltpu.async_copy(v_hbm.at[idx_c[s]], rows_v[s], sems[s]),
               pltpu.async_copy(out_hbm.at[idx_p[s]], rows_p[s], sems[s])]
        for k in range(NEG_GATHERS):
            cps.append(pltpu.async_copy(out_hbm.at[idx_n[s].at[pl.ds(k * 128, 128)]],
                                        rows_n[s].at[pl.ds(k * 128, 128)],
                                        sems[s]))
        return cps

    def compute(c, s):
        rp32 = rows_p[s]
        rn32 = rows_n[s]
        for g in range(CHUNK // 16):
            r_idx = g * 16 + iota16
            p_idx = [(g * 16 + iota16) * NEG + j for j in range(NEG)]
            zeros = jnp.zeros((16,), jnp.float32)

            def body(d, accs):
                col = jnp.broadcast_to(d, (16,))
                col2 = jnp.broadcast_to(d >> 1, (16,))
                sh = jnp.broadcast_to((d & 1) * 16, (16,))

                def bf_pick(ref, ridx):
                    u = plsc.load_gather(ref, [ridx, col2])
                    h = lax.shift_right_logical(u, sh) & 0xFFFF
                    return lax.bitcast_convert_type(
                        lax.shift_left(h, 16), jnp.float32)

                vv = plsc.load_gather(rows_v[s], [r_idx, col])
                new = [accs[0] + vv * bf_pick(rp32, r_idx)]
                for j in range(NEG):
                    new.append(accs[1 + j] + vv * bf_pick(rn32, p_idx[j]))
                return tuple(new)

            accs = lax.fori_loop(0, EMB, body, (zeros,) * (1 + NEG))
            sp[pl.ds(c * CHUNK + g * 16, 16)] = accs[0]
            for j in range(NEG):
                plsc.store_scatter(
                    sn, [(c * CHUNK + g * 16 + iota16) * NEG + j], accs[1 + j])

    cps = issue(0, 0)
    for c in range(NCHUNK):
        s = c % 2
        nxt = issue(c + 1, 1 - s) if c + 1 < NCHUNK else None
        for cp in cps:
            cp.wait()
        compute(c, s)
        cps = nxt

    pltpu.sync_copy(sp, sp_out.at[pl.ds(wid * ROWS_W, ROWS_W)])
    pltpu.sync_copy(sn, sn_out.at[pl.ds(wid * ROWS_W * NEG, ROWS_W * NEG)])


def _loss_body(sp_ref, sn_ref, out_ref):
    ps = sp_ref[...]
    ns = sn_ref[...]
    pls = jnp.minimum(ps, 0.0) - jnp.log1p(jnp.exp(-jnp.abs(ps)))
    nls = jnp.minimum(-ns, 0.0) - jnp.log1p(jnp.exp(-jnp.abs(ns)))
    out_ref[0, 0] = -(jnp.sum(pls) / BATCH) - (jnp.sum(nls) / (BATCH * NEG))


@jax.jit
def kernel(in_embed, out_embed, center, pos, neg):
    center = center.astype(jnp.int32)
    pos = pos.astype(jnp.int32)
    neg_flat = jnp.reshape(neg.astype(jnp.int32), (BATCH * NEG,))
    v = jnp.take(in_embed, center, axis=0)

    mesh = plsc.VectorSubcoreMesh(core_axis_name="c", subcore_axis_name="s")
    sc_scores = functools.partial(
        pl.kernel,
        mesh=mesh,
        compiler_params=pltpu.CompilerParams(
            needs_layout_passes=False, use_tc_tiling_on_sc=False),
        out_type=[jax.ShapeDtypeStruct((BATCH,), jnp.float32),
                  jax.ShapeDtypeStruct((BATCH * NEG,), jnp.float32)],
        scratch_types=[
            pltpu.VMEM((CHUNK,), jnp.int32), pltpu.VMEM((CHUNK,), jnp.int32),
            pltpu.VMEM((CHUNK,), jnp.int32), pltpu.VMEM((CHUNK,), jnp.int32),
            pltpu.VMEM((NEG_CH,), jnp.int32),
            pltpu.VMEM((NEG_CH,), jnp.int32),
            pltpu.VMEM((CHUNK, EMB), jnp.float32),
            pltpu.VMEM((CHUNK, EMB), jnp.float32),
            pltpu.VMEM((CHUNK, EMB // 2), jnp.int32),
            pltpu.VMEM((CHUNK, EMB // 2), jnp.int32),
            pltpu.VMEM((NEG_CH, EMB // 2), jnp.int32),
            pltpu.VMEM((NEG_CH, EMB // 2), jnp.int32),
            pltpu.VMEM((ROWS_W,), jnp.float32),
            pltpu.VMEM((ROWS_W * NEG,), jnp.float32),
            pltpu.SemaphoreType.DMA,
            pltpu.SemaphoreType.DMA,
        ],
    )(_sc_scores_body)
    out_u32 = lax.bitcast_convert_type(
        jnp.reshape(out_embed.astype(jnp.bfloat16), (VOCAB, EMB // 2, 2)),
        jnp.int32)
    sp, sn = sc_scores(pos, neg_flat, v, out_u32)

    loss = pl.pallas_call(
        _loss_body,
        out_shape=jax.ShapeDtypeStruct((1, 1), jnp.float32),
        out_specs=pl.BlockSpec(memory_space=pltpu.SMEM),
    )(jnp.reshape(sp, (BATCH // 128, 128)),
      jnp.reshape(sn, (BATCH * NEG // 128, 128)))
    return loss[0, 0]


# speedup vs baseline: 1.9690x; 1.9690x over previous
import functools
import math

import jax
import jax.numpy as jnp
from jax import lax
from jax.experimental import pallas as pl
from jax.experimental.pallas import tpu as pltpu
from jax.experimental.pallas import tpu_sc as plsc

VOCAB = 1000000
EMB = 64
BATCH = 16384
NEG = 10

NC = 2
NS = 16
NW = NC * NS
ROWS_W = BATCH // NW
CHUNK = 64
NCHUNK = ROWS_W // CHUNK
NEG_CH = CHUNK * NEG
NEG_GATHERS = NEG_CH // 128


def _sc_scores_body(pos_hbm, neg_hbm, v_hbm, out_hbm,
                    sp_out, sn_out,
                    idx_c0, idx_c1, idx_p0, idx_p1, idx_n0, idx_n1,
                    rows_v0, rows_v1, rows_p0, rows_p1, rows_n0, rows_n1,
                    sp, sn, sem0, sem1):
    wid = lax.axis_index("s") * NC + lax.axis_index("c")
    idx_c = (idx_c0, idx_c1)
    idx_p = (idx_p0, idx_p1)
    idx_n = (idx_n0, idx_n1)
    rows_v = (rows_v0, rows_v1)
    rows_p = (rows_p0, rows_p1)
    rows_n = (rows_n0, rows_n1)
    sems = (sem0, sem1)
    iota16 = lax.broadcasted_iota(jnp.int32, (16,), 0)

    def issue(c, s):
        base = wid * ROWS_W + c * CHUNK
        for g in range(CHUNK // 16):
            idx_c[s][pl.ds(g * 16, 16)] = base + g * 16 + iota16
        pltpu.sync_copy(pos_hbm.at[pl.ds(base, CHUNK)], idx_p[s])
        pltpu.sync_copy(neg_hbm.at[pl.ds(base * NEG, NEG_CH)], idx_n[s])
        cps = [pltpu.async_copy(v_hbm.at[idx_c[s]], rows_v[s], sems[s]),
               pltpu.async_copy(out_hbm.at[idx_p[s]], rows_p[s], sems[s])]
        for k in range(NEG_GATHERS):
            cps.append(pltpu.async_copy(out_hbm.at[idx_n[s].at[pl.ds(k * 128, 128)]],
                                        rows_n[s].at[pl.ds(k * 128, 128)],
                                        sems[s]))
        return cps

    def compute(c, s):
        for g in range(CHUNK // 16):
            r_idx = g * 16 + iota16
            p_idx = [(g * 16 + iota16) * NEG + j for j in range(NEG)]
            zeros = jnp.zeros((16,), jnp.float32)

            def body(d, accs):
                col = jnp.broadcast_to(d, (16,))
                vv = plsc.load_gather(rows_v[s], [r_idx, col])
                up = plsc.load_gather(rows_p[s], [r_idx, col])
                new = [accs[0] + vv * up]
                for j in range(NEG):
                    un = plsc.load_gather(rows_n[s], [p_idx[j], col])
                    new.append(accs[1 + j] + vv * un)
                return tuple(new)

            accs = lax.fori_loop(0, EMB, body, (zeros,) * (1 + NEG))
            sp[pl.ds(c * CHUNK + g * 16, 16)] = accs[0]
            for j in range(NEG):
                plsc.store_scatter(
                    sn, [(c * CHUNK + g * 16 + iota16) * NEG + j], accs[1 + j])

    cps = issue(0, 0)
    for c in range(NCHUNK):
        s = c % 2
        nxt = issue(c + 1, 1 - s) if c + 1 < NCHUNK else None
        for cp in cps:
            cp.wait()
        compute(c, s)
        cps = nxt

    pltpu.sync_copy(sp, sp_out.at[pl.ds(wid * ROWS_W, ROWS_W)])
    pltpu.sync_copy(sn, sn_out.at[pl.ds(wid * ROWS_W * NEG, ROWS_W * NEG)])


def _loss_body(sp_ref, sn_ref, out_ref):
    ps = sp_ref[...]
    ns = sn_ref[...]
    pls = jnp.minimum(ps, 0.0) - jnp.log1p(jnp.exp(-jnp.abs(ps)))
    nls = jnp.minimum(-ns, 0.0) - jnp.log1p(jnp.exp(-jnp.abs(ns)))
    out_ref[0, 0] = -(jnp.sum(pls) / BATCH) - (jnp.sum(nls) / (BATCH * NEG))


@jax.jit
def kernel(in_embed, out_embed, center, pos, neg):
    center = center.astype(jnp.int32)
    pos = pos.astype(jnp.int32)
    neg_flat = jnp.reshape(neg.astype(jnp.int32), (BATCH * NEG,))
    v = jnp.take(in_embed, center, axis=0)

    mesh = plsc.VectorSubcoreMesh(core_axis_name="c", subcore_axis_name="s")
    sc_scores = functools.partial(
        pl.kernel,
        mesh=mesh,
        compiler_params=pltpu.CompilerParams(
            needs_layout_passes=False, use_tc_tiling_on_sc=False),
        out_type=[jax.ShapeDtypeStruct((BATCH,), jnp.float32),
                  jax.ShapeDtypeStruct((BATCH * NEG,), jnp.float32)],
        scratch_types=[
            pltpu.VMEM((CHUNK,), jnp.int32), pltpu.VMEM((CHUNK,), jnp.int32),
            pltpu.VMEM((CHUNK,), jnp.int32), pltpu.VMEM((CHUNK,), jnp.int32),
            pltpu.VMEM((NEG_CH,), jnp.int32),
            pltpu.VMEM((NEG_CH,), jnp.int32),
            pltpu.VMEM((CHUNK, EMB), jnp.float32),
            pltpu.VMEM((CHUNK, EMB), jnp.float32),
            pltpu.VMEM((CHUNK, EMB), jnp.float32),
            pltpu.VMEM((CHUNK, EMB), jnp.float32),
            pltpu.VMEM((NEG_CH, EMB), jnp.float32),
            pltpu.VMEM((NEG_CH, EMB), jnp.float32),
            pltpu.VMEM((ROWS_W,), jnp.float32),
            pltpu.VMEM((ROWS_W * NEG,), jnp.float32),
            pltpu.SemaphoreType.DMA,
            pltpu.SemaphoreType.DMA,
        ],
    )(_sc_scores_body)
    sp, sn = sc_scores(pos, neg_flat, v, out_embed)

    loss = pl.pallas_call(
        _loss_body,
        out_shape=jax.ShapeDtypeStruct((1, 1), jnp.float32),
        out_specs=pl.BlockSpec(memory_space=pltpu.SMEM),
    )(jnp.reshape(sp, (BATCH // 128, 128)),
      jnp.reshape(sn, (BATCH * NEG // 128, 128)))
    return loss[0, 0]
